# Initial kernel scaffold; baseline (speedup 1.0000x reference)
import jax
import jax.numpy as jnp
from jax.experimental import pallas as pl


def kernel(x):
    raise NotImplementedError("write your pallas kernel here")


# speedup vs baseline: 1.3957x; 1.3957x over previous
import numpy as np
import jax
import jax.numpy as jnp
from jax import lax
from jax.experimental import pallas as pl
from jax.experimental.pallas import tpu as pltpu

_B, _C, _H, _W = 8, 96, 224, 224
_BS = 7
_HM, _WM = _H - (_BS - 1), _W - (_BS - 1)
_NIMG = _B * _C
_LAT = _HM * _WM
_N = _NIMG * _LAT
_COUNTM = _B * _C * _H * _W

_KEEP_RATE = max(1.0 - 0.5 / 20000.0 * 1, 1.0 - 0.5)
_GAMMA = np.float32((1.0 - _KEEP_RATE) / _BS**2 * _W**2 / (_W - _BS + 1) ** 2)


def _np_threefry2x32(ks, x0, x1):
    ks0, ks1 = np.uint32(ks[0]), np.uint32(ks[1])
    ks2 = ks0 ^ ks1 ^ np.uint32(0x1BD11BDA)
    x0 = (x0 + ks0).astype(np.uint32)
    x1 = (x1 + ks1).astype(np.uint32)
    rots = [(13, 15, 26, 6), (17, 29, 16, 24)]
    ksched = [(ks1, ks2), (ks2, ks0), (ks0, ks1), (ks1, ks2), (ks2, ks0)]
    for i in range(5):
        for r in rots[i % 2]:
            x0 = (x0 + x1).astype(np.uint32)
            x1 = ((x1 << np.uint32(r)) | (x1 >> np.uint32(32 - r))).astype(np.uint32)
            x1 = (x1 ^ x0).astype(np.uint32)
        a, b = ksched[i]
        x0 = (x0 + a).astype(np.uint32)
        x1 = (x1 + b + np.uint32(i + 1)).astype(np.uint32)
    return x0, x1


_FK0, _FK1 = _np_threefry2x32(
    (np.uint32(0), np.uint32(0)), np.array([0], np.uint32), np.array([1], np.uint32)
)
_KS0 = int(np.int32(np.uint32(_FK0[0])))
_KS1 = int(np.int32(np.uint32(_FK1[0])))
_KS2 = int(np.int32(np.uint32(_FK0[0]) ^ np.uint32(_FK1[0]) ^ np.uint32(0x1BD11BDA)))

_ROTS = ((13, 15, 26, 6), (17, 29, 16, 24))
_KSCHED = ((_KS1, _KS2), (_KS2, _KS0), (_KS0, _KS1), (_KS1, _KS2), (_KS2, _KS0))


def _rotl(x, r):
    return lax.shift_left(x, np.int32(r)) | lax.shift_right_logical(x, np.int32(32 - r))


def _threefry_bits(x1):
    x0 = jnp.full(x1.shape, _KS0, jnp.int32)
    x1 = x1 + np.int32(_KS1)
    for i in range(5):
        for r in _ROTS[i % 2]:
            x0 = x0 + x1
            x1 = _rotl(x1, r)
            x1 = x1 ^ x0
        a, b = _KSCHED[i]
        x0 = x0 + np.int32(a)
        x1 = x1 + np.int32(np.int32(b) + np.int32(i + 1))
    return x0 ^ x1


def _or_window7(x, axis):
    x = x | jnp.roll(x, 1, axis)
    x = x | jnp.roll(x, 2, axis)
    x = x | jnp.roll(x, 3, axis)
    return x


def _mask_kernel(mask_ref, count_ref):
    img = pl.program_id(0)
    h = lax.broadcasted_iota(jnp.int32, (_H, _W), 0)
    w = lax.broadcasted_iota(jnp.int32, (_H, _W), 1)
    valid = (h < _HM) & (w < _WM)
    idx = img * np.int32(_LAT) + h * np.int32(_WM) + w
    bits = _threefry_bits(idx)
    fbits = lax.shift_right_logical(bits, np.int32(9)) | np.int32(0x3F800000)
    u = lax.bitcast_convert_type(fbits, jnp.float32) - np.float32(1.0)
    seed = jnp.where((u < _GAMMA) & valid, np.int32(1), np.int32(0))
    dil = _or_window7(_or_window7(seed, 1), 0)
    mask_ref[0] = dil.astype(jnp.float32)

    @pl.when(img == 0)
    def _init():
        count_ref[0, 0] = 0

    count_ref[0, 0] += jnp.sum(dil)


def _apply_kernel(x_ref, mask_ref, count_ref, out_ref):
    cnt = count_ref[0, 0]
    scale = np.float32(_COUNTM) / (np.int32(_COUNTM) - cnt).astype(jnp.float32)
    drop = mask_ref[...] != 0
    out_ref[...] = x_ref[...] * jnp.where(drop, np.float32(0.0), scale)


_APPLY_ROWS = 4


def _dropblock_impl(x):
    xr = x.reshape(_NIMG, _H, _W)
    mask, count = pl.pallas_call(
        _mask_kernel,
        grid=(_NIMG,),
        out_specs=[
            pl.BlockSpec((1, _H, _W), lambda i: (i, 0, 0)),
            pl.BlockSpec(memory_space=pltpu.SMEM),
        ],
        out_shape=[
            jax.ShapeDtypeStruct((_NIMG, _H, _W), jnp.float32),
            jax.ShapeDtypeStruct((1, 1), jnp.int32),
        ],
    )()
    out = pl.pallas_call(
        _apply_kernel,
        grid=(_NIMG // _APPLY_ROWS,),
        in_specs=[
            pl.BlockSpec((_APPLY_ROWS, _H, _W), lambda i: (i, 0, 0)),
            pl.BlockSpec((_APPLY_ROWS, _H, _W), lambda i: (i, 0, 0)),
            pl.BlockSpec(memory_space=pltpu.SMEM),
        ],
        out_specs=pl.BlockSpec((_APPLY_ROWS, _H, _W), lambda i: (i, 0, 0)),
        out_shape=jax.ShapeDtypeStruct((_NIMG, _H, _W), jnp.float32),
    )(xr, mask, count)
    return out.reshape(_B, _C, _H, _W)


def kernel(x):
    return _dropblock_impl(x)


# speedup vs baseline: 1.5970x; 1.1442x over previous
import numpy as np
import jax
import jax.numpy as jnp
from jax import lax
from jax.experimental import pallas as pl
from jax.experimental.pallas import tpu as pltpu

_B, _C, _H, _W = 8, 96, 224, 224
_BS = 7
_HM, _WM = _H - (_BS - 1), _W - (_BS - 1)
_NIMG = _B * _C
_LAT = _HM * _WM
_N = _NIMG * _LAT
_COUNTM = _B * _C * _H * _W

_KEEP_RATE = max(1.0 - 0.5 / 20000.0 * 1, 1.0 - 0.5)
_GAMMA = np.float32((1.0 - _KEEP_RATE) / _BS**2 * _W**2 / (_W - _BS + 1) ** 2)
_MTHRESH = int(np.ceil(np.float64(_GAMMA) * 2.0**23))


def _np_threefry2x32(ks, x0, x1):
    ks0, ks1 = np.uint32(ks[0]), np.uint32(ks[1])
    ks2 = ks0 ^ ks1 ^ np.uint32(0x1BD11BDA)
    x0 = (x0 + ks0).astype(np.uint32)
    x1 = (x1 + ks1).astype(np.uint32)
    rots = [(13, 15, 26, 6), (17, 29, 16, 24)]
    ksched = [(ks1, ks2), (ks2, ks0), (ks0, ks1), (ks1, ks2), (ks2, ks0)]
    for i in range(5):
        for r in rots[i % 2]:
            x0 = (x0 + x1).astype(np.uint32)
            x1 = ((x1 << np.uint32(r)) | (x1 >> np.uint32(32 - r))).astype(np.uint32)
            x1 = (x1 ^ x0).astype(np.uint32)
        a, b = ksched[i]
        x0 = (x0 + a).astype(np.uint32)
        x1 = (x1 + b + np.uint32(i + 1)).astype(np.uint32)
    return x0, x1


_FK0, _FK1 = _np_threefry2x32(
    (np.uint32(0), np.uint32(0)), np.array([0], np.uint32), np.array([1], np.uint32)
)
_KS0 = int(np.int32(np.uint32(_FK0[0])))
_KS1 = int(np.int32(np.uint32(_FK1[0])))
_KS2 = int(np.int32(np.uint32(_FK0[0]) ^ np.uint32(_FK1[0]) ^ np.uint32(0x1BD11BDA)))

_ROTS = ((13, 15, 26, 6), (17, 29, 16, 24))
_KSCHED = ((_KS1, _KS2), (_KS2, _KS0), (_KS0, _KS1), (_KS1, _KS2), (_KS2, _KS0))


def _rotl(x, r):
    return lax.shift_left(x, np.int32(r)) | lax.shift_right_logical(x, np.int32(32 - r))


def _threefry_bits(x1):
    x0 = jnp.full(x1.shape, _KS0, jnp.int32)
    x1 = x1 + np.int32(_KS1)
    for i in range(5):
        for r in _ROTS[i % 2]:
            x0 = x0 + x1
            x1 = _rotl(x1, r)
            x1 = x1 ^ x0
        a, b = _KSCHED[i]
        x0 = x0 + np.int32(a)
        x1 = x1 + np.int32(np.int32(b) + np.int32(i + 1))
    return x0 ^ x1


def _or_window7(x, axis):
    x = x | jnp.roll(x, 1, axis)
    x = x | jnp.roll(x, 2, axis)
    x = x | jnp.roll(x, 3, axis)
    return x


_MASK_ROWS = 4


def _mask_kernel(mask_ref, count_ref):
    step = pl.program_id(0)
    a = lax.broadcasted_iota(jnp.int32, (_MASK_ROWS, _H, _W), 0)
    h = lax.broadcasted_iota(jnp.int32, (_MASK_ROWS, _H, _W), 1)
    w = lax.broadcasted_iota(jnp.int32, (_MASK_ROWS, _H, _W), 2)
    valid = (h < _HM) & (w < _WM)
    idx = (step * np.int32(_MASK_ROWS) + a) * np.int32(_LAT) + h * np.int32(_WM) + w
    bits = _threefry_bits(idx)
    m = lax.shift_right_logical(bits, np.int32(9))
    seed = jnp.where((m < _MTHRESH) & valid, np.int32(1), np.int32(0))
    dil = _or_window7(_or_window7(seed, 2), 1)
    mask_ref[...] = dil.astype(jnp.int8)

    @pl.when(step == 0)
    def _init():
        count_ref[0, 0] = 0

    count_ref[0, 0] += jnp.sum(dil)


def _apply_kernel(x_ref, mask_ref, count_ref, out_ref):
    cnt = count_ref[0, 0]
    scale = np.float32(_COUNTM) / (np.int32(_COUNTM) - cnt).astype(jnp.float32)
    keep = np.float32(1.0) - mask_ref[...].astype(jnp.float32)
    out_ref[...] = x_ref[...] * (scale * keep)


_APPLY_ROWS = 4


def _dropblock_impl(x):
    xr = x.reshape(_NIMG, _H, _W)
    mask, count = pl.pallas_call(
        _mask_kernel,
        grid=(_NIMG // _MASK_ROWS,),
        out_specs=[
            pl.BlockSpec((_MASK_ROWS, _H, _W), lambda i: (i, 0, 0)),
            pl.BlockSpec(memory_space=pltpu.SMEM),
        ],
        out_shape=[
            jax.ShapeDtypeStruct((_NIMG, _H, _W), jnp.int8),
            jax.ShapeDtypeStruct((1, 1), jnp.int32),
        ],
    )()
    out = pl.pallas_call(
        _apply_kernel,
        grid=(_NIMG // _APPLY_ROWS,),
        in_specs=[
            pl.BlockSpec((_APPLY_ROWS, _H, _W), lambda i: (i, 0, 0)),
            pl.BlockSpec((_APPLY_ROWS, _H, _W), lambda i: (i, 0, 0)),
            pl.BlockSpec(memory_space=pltpu.SMEM),
        ],
        out_specs=pl.BlockSpec((_APPLY_ROWS, _H, _W), lambda i: (i, 0, 0)),
        out_shape=jax.ShapeDtypeStruct((_NIMG, _H, _W), jnp.float32),
    )(xr, mask, count)
    return out.reshape(_B, _C, _H, _W)


def kernel(x):
    return _dropblock_impl(x)


# speedup vs baseline: 1.6526x; 1.0348x over previous
import numpy as np
import jax
import jax.numpy as jnp
from jax import lax
from jax.experimental import pallas as pl
from jax.experimental.pallas import tpu as pltpu

_B, _C, _H, _W = 8, 96, 224, 224
_BS = 7
_HM, _WM = _H - (_BS - 1), _W - (_BS - 1)
_NIMG = _B * _C
_LAT = _HM * _WM
_COUNTM = _B * _C * _H * _W

_KEEP_RATE = max(1.0 - 0.5 / 20000.0 * 1, 1.0 - 0.5)
_GAMMA = np.float32((1.0 - _KEEP_RATE) / _BS**2 * _W**2 / (_W - _BS + 1) ** 2)
_MTHRESH = int(np.ceil(np.float64(_GAMMA) * 2.0**23))


def _np_threefry2x32(ks, x0, x1):
    ks0, ks1 = np.uint32(ks[0]), np.uint32(ks[1])
    ks2 = ks0 ^ ks1 ^ np.uint32(0x1BD11BDA)
    x0 = (x0 + ks0).astype(np.uint32)
    x1 = (x1 + ks1).astype(np.uint32)
    rots = [(13, 15, 26, 6), (17, 29, 16, 24)]
    ksched = [(ks1, ks2), (ks2, ks0), (ks0, ks1), (ks1, ks2), (ks2, ks0)]
    for i in range(5):
        for r in rots[i % 2]:
            x0 = (x0 + x1).astype(np.uint32)
            x1 = ((x1 << np.uint32(r)) | (x1 >> np.uint32(32 - r))).astype(np.uint32)
            x1 = (x1 ^ x0).astype(np.uint32)
        a, b = ksched[i]
        x0 = (x0 + a).astype(np.uint32)
        x1 = (x1 + b + np.uint32(i + 1)).astype(np.uint32)
    return x0, x1


_FK0, _FK1 = _np_threefry2x32(
    (np.uint32(0), np.uint32(0)), np.array([0], np.uint32), np.array([1], np.uint32)
)
_KS0 = int(np.int32(np.uint32(_FK0[0])))
_KS1 = int(np.int32(np.uint32(_FK1[0])))
_KS2 = int(np.int32(np.uint32(_FK0[0]) ^ np.uint32(_FK1[0]) ^ np.uint32(0x1BD11BDA)))

_ROTS = ((13, 15, 26, 6), (17, 29, 16, 24))
_KSCHED = ((_KS1, _KS2), (_KS2, _KS0), (_KS0, _KS1), (_KS1, _KS2), (_KS2, _KS0))


def _rotl(x, r):
    return lax.shift_left(x, np.int32(r)) | lax.shift_right_logical(x, np.int32(32 - r))


def _threefry_bits(x1):
    x0 = jnp.full(x1.shape, _KS0, jnp.int32)
    x1 = x1 + np.int32(_KS1)
    for i in range(5):
        for r in _ROTS[i % 2]:
            x0 = x0 + x1
            x1 = _rotl(x1, r)
            x1 = x1 ^ x0
        a, b = _KSCHED[i]
        x0 = x0 + np.int32(a)
        x1 = x1 + np.int32(np.int32(b) + np.int32(i + 1))
    return x0 ^ x1


_APPLY_ROWS = 4
_CHUNK = 32
_NCHUNK = _H // _CHUNK


def _seed_chunk(img, r):
    h = lax.broadcasted_iota(jnp.int32, (_CHUNK, _W), 0) + r * np.int32(_CHUNK)
    w = lax.broadcasted_iota(jnp.int32, (_CHUNK, _W), 1)
    valid = (h < _HM) & (w < _WM)
    idx = img * np.int32(_LAT) + h * np.int32(_WM) + w
    bits = _threefry_bits(idx)
    m = lax.shift_right_logical(bits, np.int32(9))
    isseed = (m < _MTHRESH) & valid
    word = lax.shift_left(h, np.int32(14)) + lax.shift_left(w, np.int32(6)) + 1
    return jnp.where(isseed, word, np.int32(0))


def _seed_kernel(code_ref, tot_ref):
    img = pl.program_id(0)

    @pl.when(img == 0)
    def _init():
        tot_ref[0, 0] = 0

    def body(r, acc):
        return acc + _seed_chunk(img, r)

    acc = lax.fori_loop(0, _NCHUNK, body, jnp.zeros((_CHUNK, _W), jnp.int32))
    s = jnp.sum(acc)
    code_ref[0, 0, 0] = s
    tot_ref[0, 0] += s & np.int32(63)


def _apply_kernel(x_ref, code_ref, tot_ref, out_ref):
    tot = tot_ref[0, 0]
    dropped = (np.int32(_BS * _BS) * tot).astype(jnp.float32)
    scale = np.float32(_COUNTM) / (np.float32(_COUNTM) - dropped)
    codes = [code_ref[0, 0, i] for i in range(_APPLY_ROWS)]
    tilecnt = codes[0] & 63
    for s in codes[1:]:
        tilecnt += s & np.int32(63)

    @pl.when(tilecnt == 0)
    def _fast():
        out_ref[...] = x_ref[...] * scale

    @pl.when(tilecnt > 0)
    def _slow():
        oh = lax.broadcasted_iota(jnp.int32, (_H, _W), 0)
        ow = lax.broadcasted_iota(jnp.int32, (_H, _W), 1)
        for i in range(_APPLY_ROWS):
            s = codes[i]
            cnt = s & np.int32(63)
            h0 = jnp.where(cnt > 0, lax.shift_right_logical(s, np.int32(14)), np.int32(300))
            w0 = lax.shift_right_logical(s, np.int32(6)) & np.int32(255)
            drop = (oh >= h0) & (oh < h0 + np.int32(_BS)) & (ow >= w0) & (ow < w0 + np.int32(_BS))
            out_ref[i] = x_ref[i] * jnp.where(drop, np.float32(0.0), scale)


def _dropblock_impl(x):
    xr = x.reshape(_NIMG, _H, _W)
    code, tot = pl.pallas_call(
        _seed_kernel,
        grid=(_NIMG,),
        out_specs=[
            pl.BlockSpec((1, 1, 1), lambda i: (i, 0, 0), memory_space=pltpu.SMEM),
            pl.BlockSpec(memory_space=pltpu.SMEM),
        ],
        out_shape=[
            jax.ShapeDtypeStruct((_NIMG, 1, 1), jnp.int32),
            jax.ShapeDtypeStruct((1, 1), jnp.int32),
        ],
    )()
    code = code.reshape(_NIMG // _APPLY_ROWS, 1, _APPLY_ROWS)
    out = pl.pallas_call(
        _apply_kernel,
        grid=(_NIMG // _APPLY_ROWS,),
        in_specs=[
            pl.BlockSpec((_APPLY_ROWS, _H, _W), lambda i: (i, 0, 0)),
            pl.BlockSpec((1, 1, _APPLY_ROWS), lambda i: (i, 0, 0), memory_space=pltpu.SMEM),
            pl.BlockSpec(memory_space=pltpu.SMEM),
        ],
        out_specs=pl.BlockSpec((_APPLY_ROWS, _H, _W), lambda i: (i, 0, 0)),
        out_shape=jax.ShapeDtypeStruct((_NIMG, _H, _W), jnp.float32),
    )(xr, code, tot)
    return out.reshape(_B, _C, _H, _W)


def kernel(x):
    return _dropblock_impl(x)


# speedup vs baseline: 1.6863x; 1.0203x over previous
import numpy as np
import jax
import jax.numpy as jnp
from jax import lax
from jax.experimental import pallas as pl
from jax.experimental.pallas import tpu as pltpu

_B, _C, _H, _W = 8, 96, 224, 224
_BS = 7
_HM, _WM = _H - (_BS - 1), _W - (_BS - 1)
_NIMG = _B * _C
_LAT = _HM * _WM
_COUNTM = _B * _C * _H * _W

_KEEP_RATE = max(1.0 - 0.5 / 20000.0 * 1, 1.0 - 0.5)
_GAMMA = np.float32((1.0 - _KEEP_RATE) / _BS**2 * _W**2 / (_W - _BS + 1) ** 2)
_MTHRESH = int(np.ceil(np.float64(_GAMMA) * 2.0**23))


def _np_threefry2x32(ks, x0, x1):
    ks0, ks1 = np.uint32(ks[0]), np.uint32(ks[1])
    ks2 = ks0 ^ ks1 ^ np.uint32(0x1BD11BDA)
    x0 = (x0 + ks0).astype(np.uint32)
    x1 = (x1 + ks1).astype(np.uint32)
    rots = [(13, 15, 26, 6), (17, 29, 16, 24)]
    ksched = [(ks1, ks2), (ks2, ks0), (ks0, ks1), (ks1, ks2), (ks2, ks0)]
    for i in range(5):
        for r in rots[i % 2]:
            x0 = (x0 + x1).astype(np.uint32)
            x1 = ((x1 << np.uint32(r)) | (x1 >> np.uint32(32 - r))).astype(np.uint32)
            x1 = (x1 ^ x0).astype(np.uint32)
        a, b = ksched[i]
        x0 = (x0 + a).astype(np.uint32)
        x1 = (x1 + b + np.uint32(i + 1)).astype(np.uint32)
    return x0, x1


_FK0, _FK1 = _np_threefry2x32(
    (np.uint32(0), np.uint32(0)), np.array([0], np.uint32), np.array([1], np.uint32)
)
_KS0 = int(np.int32(np.uint32(_FK0[0])))
_KS1 = int(np.int32(np.uint32(_FK1[0])))
_KS2 = int(np.int32(np.uint32(_FK0[0]) ^ np.uint32(_FK1[0]) ^ np.uint32(0x1BD11BDA)))

_ROTS = ((13, 15, 26, 6), (17, 29, 16, 24))
_KSCHED = ((_KS1, _KS2), (_KS2, _KS0), (_KS0, _KS1), (_KS1, _KS2), (_KS2, _KS0))


def _rotl(x, r):
    return lax.shift_left(x, np.int32(r)) | lax.shift_right_logical(x, np.int32(32 - r))


def _threefry_bits(x1):
    x0 = jnp.full(x1.shape, _KS0, jnp.int32)
    x1 = x1 + np.int32(_KS1)
    for i in range(5):
        for r in _ROTS[i % 2]:
            x0 = x0 + x1
            x1 = _rotl(x1, r)
            x1 = x1 ^ x0
        a, b = _KSCHED[i]
        x0 = x0 + np.int32(a)
        x1 = x1 + np.int32(np.int32(b) + np.int32(i + 1))
    return x0 ^ x1


_APPLY_ROWS = 4
_CHUNK = 56
_NCHUNK = _H // _CHUNK


def _seed_kernel(code_ref, tot_ref):
    img = pl.program_id(0)

    @pl.when(img == 0)
    def _init():
        tot_ref[0, 0] = 0

    hl = lax.broadcasted_iota(jnp.int32, (_CHUNK, _W), 0)
    w = lax.broadcasted_iota(jnp.int32, (_CHUNK, _W), 1)
    wvalid = w < _WM
    qbase = hl * np.int32(_WM) + w
    wbase = lax.shift_left(hl, np.int32(14)) + lax.shift_left(w, np.int32(6)) + 1

    def body(r, acc):
        h_off = r * np.int32(_CHUNK)
        idx = (img * np.int32(_LAT) + h_off * np.int32(_WM)) + qbase
        bits = _threefry_bits(idx)
        m = lax.shift_right_logical(bits, np.int32(9))
        isseed = (m < _MTHRESH) & wvalid & (hl < _HM - h_off)
        word = wbase + lax.shift_left(h_off, np.int32(14))
        return acc + jnp.where(isseed, word, np.int32(0))

    acc = lax.fori_loop(0, _NCHUNK, body, jnp.zeros((_CHUNK, _W), jnp.int32))
    s = jnp.sum(acc)
    code_ref[0, 0, 0] = s
    tot_ref[0, 0] += s & np.int32(63)


def _apply_kernel(x_ref, code_ref, tot_ref, out_ref):
    tot = tot_ref[0, 0]
    dropped = (np.int32(_BS * _BS) * tot).astype(jnp.float32)
    scale = np.float32(_COUNTM) / (np.float32(_COUNTM) - dropped)
    codes = [code_ref[0, 0, i] for i in range(_APPLY_ROWS)]
    tilecnt = codes[0] & 63
    for s in codes[1:]:
        tilecnt += s & np.int32(63)

    @pl.when(tilecnt == 0)
    def _fast():
        out_ref[...] = x_ref[...] * scale

    @pl.when(tilecnt > 0)
    def _slow():
        oh = lax.broadcasted_iota(jnp.int32, (_H, _W), 0)
        ow = lax.broadcasted_iota(jnp.int32, (_H, _W), 1)
        for i in range(_APPLY_ROWS):
            s = codes[i]
            cnt = s & np.int32(63)
            h0 = jnp.where(cnt > 0, lax.shift_right_logical(s, np.int32(14)), np.int32(300))
            w0 = lax.shift_right_logical(s, np.int32(6)) & np.int32(255)
            drop = (oh >= h0) & (oh < h0 + np.int32(_BS)) & (ow >= w0) & (ow < w0 + np.int32(_BS))
            out_ref[i] = x_ref[i] * jnp.where(drop, np.float32(0.0), scale)


def _dropblock_impl(x):
    xr = x.reshape(_NIMG, _H, _W)
    code, tot = pl.pallas_call(
        _seed_kernel,
        grid=(_NIMG,),
        out_specs=[
            pl.BlockSpec((1, 1, 1), lambda i: (i, 0, 0), memory_space=pltpu.SMEM),
            pl.BlockSpec(memory_space=pltpu.SMEM),
        ],
        out_shape=[
            jax.ShapeDtypeStruct((_NIMG, 1, 1), jnp.int32),
            jax.ShapeDtypeStruct((1, 1), jnp.int32),
        ],
    )()
    code = code.reshape(_NIMG // _APPLY_ROWS, 1, _APPLY_ROWS)
    out = pl.pallas_call(
        _apply_kernel,
        grid=(_NIMG // _APPLY_ROWS,),
        in_specs=[
            pl.BlockSpec((_APPLY_ROWS, _H, _W), lambda i: (i, 0, 0)),
            pl.BlockSpec((1, 1, _APPLY_ROWS), lambda i: (i, 0, 0), memory_space=pltpu.SMEM),
            pl.BlockSpec(memory_space=pltpu.SMEM),
        ],
        out_specs=pl.BlockSpec((_APPLY_ROWS, _H, _W), lambda i: (i, 0, 0)),
        out_shape=jax.ShapeDtypeStruct((_NIMG, _H, _W), jnp.float32),
    )(xr, code, tot)
    return out.reshape(_B, _C, _H, _W)


def kernel(x):
    return _dropblock_impl(x)


# speedup vs baseline: 1.9583x; 1.1613x over previous
import numpy as np
import jax
import jax.numpy as jnp
from jax import lax
from jax.experimental import pallas as pl
from jax.experimental.pallas import tpu as pltpu

_B, _C, _H, _W = 8, 96, 224, 224
_BS = 7
_HM, _WM = _H - (_BS - 1), _W - (_BS - 1)
_NIMG = _B * _C
_LAT = _HM * _WM
_COUNTM = _B * _C * _H * _W

_KEEP_RATE = max(1.0 - 0.5 / 20000.0 * 1, 1.0 - 0.5)
_GAMMA = np.float32((1.0 - _KEEP_RATE) / _BS**2 * _W**2 / (_W - _BS + 1) ** 2)
_MTHRESH = int(np.ceil(np.float64(_GAMMA) * 2.0**23))


def _np_threefry2x32(ks, x0, x1):
    ks0, ks1 = np.uint32(ks[0]), np.uint32(ks[1])
    ks2 = ks0 ^ ks1 ^ np.uint32(0x1BD11BDA)
    x0 = (x0 + ks0).astype(np.uint32)
    x1 = (x1 + ks1).astype(np.uint32)
    rots = [(13, 15, 26, 6), (17, 29, 16, 24)]
    ksched = [(ks1, ks2), (ks2, ks0), (ks0, ks1), (ks1, ks2), (ks2, ks0)]
    for i in range(5):
        for r in rots[i % 2]:
            x0 = (x0 + x1).astype(np.uint32)
            x1 = ((x1 << np.uint32(r)) | (x1 >> np.uint32(32 - r))).astype(np.uint32)
            x1 = (x1 ^ x0).astype(np.uint32)
        a, b = ksched[i]
        x0 = (x0 + a).astype(np.uint32)
        x1 = (x1 + b + np.uint32(i + 1)).astype(np.uint32)
    return x0, x1


_FK0, _FK1 = _np_threefry2x32(
    (np.uint32(0), np.uint32(0)), np.array([0], np.uint32), np.array([1], np.uint32)
)
_KS0 = int(np.int32(np.uint32(_FK0[0])))
_KS1 = int(np.int32(np.uint32(_FK1[0])))
_KS2 = int(np.int32(np.uint32(_FK0[0]) ^ np.uint32(_FK1[0]) ^ np.uint32(0x1BD11BDA)))

_ROTS = ((13, 15, 26, 6), (17, 29, 16, 24))
_KSCHED = ((_KS1, _KS2), (_KS2, _KS0), (_KS0, _KS1), (_KS1, _KS2), (_KS2, _KS0))


def _rotl(x, r):
    return lax.shift_left(x, np.int32(r)) | lax.shift_right_logical(x, np.int32(32 - r))


def _threefry_bits(x1):
    x0 = jnp.full(x1.shape, _KS0, jnp.int32)
    x1 = x1 + np.int32(_KS1)
    for i in range(5):
        for r in _ROTS[i % 2]:
            x0 = x0 + x1
            x1 = _rotl(x1, r)
            x1 = x1 ^ x0
        a, b = _KSCHED[i]
        x0 = x0 + np.int32(a)
        x1 = x1 + np.int32(np.int32(b) + np.int32(i + 1))
    return x0 ^ x1


_APPLY_ROWS = 4
_SEED_IMGS = 4
_CROWS = 96
_CSIZE = _CROWS * 128
_NCHUNK = -(-_LAT // _CSIZE)


def _seed_kernel(code_ref, tot_ref):
    step = pl.program_id(0)

    @pl.when(step == 0)
    def _init():
        tot_ref[0, 0] = 0

    rr = lax.broadcasted_iota(jnp.int32, (_CROWS, 128), 0)
    l = lax.broadcasted_iota(jnp.int32, (_CROWS, 128), 1)
    qbase = rr * np.int32(128) + l
    wordbase = lax.shift_left(qbase, np.int32(6)) + 1

    for a in range(_SEED_IMGS):
        img = step * np.int32(_SEED_IMGS) + np.int32(a)
        base = img * np.int32(_LAT)

        def body(r, acc):
            off = r * np.int32(_CSIZE)
            bits = _threefry_bits(base + off + qbase)
            m = lax.shift_right_logical(bits, np.int32(9))
            isseed = (m < _MTHRESH) & (qbase < _LAT - off)
            word = wordbase + lax.shift_left(off, np.int32(6))
            return acc + jnp.where(isseed, word, np.int32(0))

        acc = lax.fori_loop(0, _NCHUNK, body, jnp.zeros((_CROWS, 128), jnp.int32))
        sa = jnp.sum(acc)
        code_ref[0, 0, a] = sa
        tot_ref[0, 0] += sa & np.int32(63)


def _apply_kernel(x_ref, code_ref, tot_ref, out_ref):
    tot = tot_ref[0, 0]
    dropped = (np.int32(_BS * _BS) * tot).astype(jnp.float32)
    scale = np.float32(_COUNTM) / (np.float32(_COUNTM) - dropped)
    codes = [code_ref[0, 0, i] for i in range(_APPLY_ROWS)]
    tilecnt = codes[0] & 63
    for s in codes[1:]:
        tilecnt += s & np.int32(63)

    @pl.when(tilecnt == 0)
    def _fast():
        out_ref[...] = x_ref[...] * scale

    @pl.when(tilecnt > 0)
    def _slow():
        oh = lax.broadcasted_iota(jnp.int32, (_H, _W), 0)
        ow = lax.broadcasted_iota(jnp.int32, (_H, _W), 1)
        for i in range(_APPLY_ROWS):
            s = codes[i]
            cnt = s & np.int32(63)
            q = lax.shift_right_logical(s, np.int32(6))
            hq = ((q.astype(jnp.float32) + np.float32(0.5)) * np.float32(1.0 / _WM)).astype(jnp.int32)
            h0 = jnp.where(cnt > 0, hq, np.int32(300))
            w0 = q - np.int32(_WM) * hq
            drop = (oh >= h0) & (oh < h0 + np.int32(_BS)) & (ow >= w0) & (ow < w0 + np.int32(_BS))
            out_ref[i] = x_ref[i] * jnp.where(drop, np.float32(0.0), scale)


def _dropblock_impl(x):
    xr = x.reshape(_NIMG, _H, _W)
    code, tot = pl.pallas_call(
        _seed_kernel,
        grid=(_NIMG // _SEED_IMGS,),
        out_specs=[
            pl.BlockSpec((1, 1, _SEED_IMGS), lambda i: (i, 0, 0), memory_space=pltpu.SMEM),
            pl.BlockSpec(memory_space=pltpu.SMEM),
        ],
        out_shape=[
            jax.ShapeDtypeStruct((_NIMG // _SEED_IMGS, 1, _SEED_IMGS), jnp.int32),
            jax.ShapeDtypeStruct((1, 1), jnp.int32),
        ],
    )()
    out = pl.pallas_call(
        _apply_kernel,
        grid=(_NIMG // _APPLY_ROWS,),
        in_specs=[
            pl.BlockSpec((_APPLY_ROWS, _H, _W), lambda i: (i, 0, 0)),
            pl.BlockSpec((1, 1, _APPLY_ROWS), lambda i: (i, 0, 0), memory_space=pltpu.SMEM),
            pl.BlockSpec(memory_space=pltpu.SMEM),
        ],
        out_specs=pl.BlockSpec((_APPLY_ROWS, _H, _W), lambda i: (i, 0, 0)),
        out_shape=jax.ShapeDtypeStruct((_NIMG, _H, _W), jnp.float32),
    )(xr, code, tot)
    return out.reshape(_B, _C, _H, _W)


def kernel(x):
    return _dropblock_impl(x)


# speedup vs baseline: 2.1230x; 1.0841x over previous
import functools

import numpy as np
import jax
import jax.numpy as jnp
from jax import lax
from jax.experimental import pallas as pl
from jax.experimental.pallas import tpu as pltpu
from jax.experimental.pallas import tpu_sc as plsc

_B, _C, _H, _W = 8, 96, 224, 224
_BS = 7
_HM, _WM = _H - (_BS - 1), _W - (_BS - 1)
_NIMG = _B * _C
_LAT = _HM * _WM
_COUNTM = _B * _C * _H * _W

_KEEP_RATE = max(1.0 - 0.5 / 20000.0 * 1, 1.0 - 0.5)
_GAMMA = np.float32((1.0 - _KEEP_RATE) / _BS**2 * _W**2 / (_W - _BS + 1) ** 2)
_MTHRESH = int(np.ceil(np.float64(_GAMMA) * 2.0**23))


def _np_threefry2x32(ks, x0, x1):
    ks0, ks1 = np.uint32(ks[0]), np.uint32(ks[1])
    ks2 = ks0 ^ ks1 ^ np.uint32(0x1BD11BDA)
    x0 = (x0 + ks0).astype(np.uint32)
    x1 = (x1 + ks1).astype(np.uint32)
    rots = [(13, 15, 26, 6), (17, 29, 16, 24)]
    ksched = [(ks1, ks2), (ks2, ks0), (ks0, ks1), (ks1, ks2), (ks2, ks0)]
    for i in range(5):
        for r in rots[i % 2]:
            x0 = (x0 + x1).astype(np.uint32)
            x1 = ((x1 << np.uint32(r)) | (x1 >> np.uint32(32 - r))).astype(np.uint32)
            x1 = (x1 ^ x0).astype(np.uint32)
        a, b = ksched[i]
        x0 = (x0 + a).astype(np.uint32)
        x1 = (x1 + b + np.uint32(i + 1)).astype(np.uint32)
    return x0, x1


_FK0, _FK1 = _np_threefry2x32(
    (np.uint32(0), np.uint32(0)), np.array([0], np.uint32), np.array([1], np.uint32)
)
_KS0 = int(np.int32(np.uint32(_FK0[0])))
_KS1 = int(np.int32(np.uint32(_FK1[0])))
_KS2 = int(np.int32(np.uint32(_FK0[0]) ^ np.uint32(_FK1[0]) ^ np.uint32(0x1BD11BDA)))

_ROTS = ((13, 15, 26, 6), (17, 29, 16, 24))
_KSCHED = ((_KS1, _KS2), (_KS2, _KS0), (_KS0, _KS1), (_KS1, _KS2), (_KS2, _KS0))


def _rotl(x, r):
    return lax.shift_left(x, np.int32(r)) | lax.shift_right_logical(x, np.int32(32 - r))


def _threefry_bits(x1):
    x0 = jnp.full(x1.shape, _KS0, jnp.int32)
    x1 = x1 + np.int32(_KS1)
    for i in range(5):
        for r in _ROTS[i % 2]:
            x0 = x0 + x1
            x1 = _rotl(x1, r)
            x1 = x1 ^ x0
        a, b = _KSCHED[i]
        x0 = x0 + np.int32(a)
        x1 = x1 + np.int32(np.int32(b) + np.int32(i + 1))
    return x0 ^ x1


_APPLY_ROWS = 4
_SEED_IMGS = 4
_CROWS = 96
_CSIZE = _CROWS * 128
_NCHUNK = -(-_LAT // _CSIZE)


def _seed_kernel(code_ref, tot_ref):
    step = pl.program_id(0)

    @pl.when(step == 0)
    def _init():
        tot_ref[0, 0] = 0

    rr = lax.broadcasted_iota(jnp.int32, (_CROWS, 128), 0)
    l = lax.broadcasted_iota(jnp.int32, (_CROWS, 128), 1)
    qbase = rr * np.int32(128) + l
    wordbase = lax.shift_left(qbase, np.int32(6)) + 1

    for a in range(_SEED_IMGS):
        img = step * np.int32(_SEED_IMGS) + np.int32(a)
        base = img * np.int32(_LAT)

        def body(r, acc):
            off = r * np.int32(_CSIZE)
            bits = _threefry_bits(base + off + qbase)
            m = lax.shift_right_logical(bits, np.int32(9))
            isseed = (m < _MTHRESH) & (qbase < _LAT - off)
            word = wordbase + lax.shift_left(off, np.int32(6))
            return acc + jnp.where(isseed, word, np.int32(0))

        acc = lax.fori_loop(0, _NCHUNK, body, jnp.zeros((_CROWS, 128), jnp.int32))
        sa = jnp.sum(acc)
        code_ref[0, 0, a] = sa
        tot_ref[0, 0] += sa & np.int32(63)


_SC_IMG0 = 512
_SC_IMGS = _NIMG - _SC_IMG0
_SC_TILES = 32
_SC_PER_TILE = _SC_IMGS // _SC_TILES
_SC_UNROLL = 4
_SC_ITERS = -(-_LAT // (16 * _SC_UNROLL))


def _sc_seed_kernel(out_hbm, codes_v):
    wid = lax.axis_index("s") * np.int32(2) + lax.axis_index("c")
    li = lax.iota(jnp.int32, 16)
    for j in range(_SC_PER_TILE):
        img = np.int32(_SC_IMG0) + wid * np.int32(_SC_PER_TILE) + np.int32(j)
        base = img * np.int32(_LAT)

        def body(t, accs):
            qb = t * np.int32(16 * _SC_UNROLL)
            new = []
            for u in range(_SC_UNROLL):
                q = qb + np.int32(u * 16) + li
                bits = _threefry_bits(base + q)
                m = lax.shift_right_logical(bits, np.int32(9))
                isseed = (m < _MTHRESH) & (q < np.int32(_LAT))
                word = lax.shift_left(q, np.int32(6)) + 1
                new.append(accs[u] + jnp.where(isseed, word, np.int32(0)))
            return tuple(new)

        z = jnp.zeros((16,), jnp.int32)
        accs = lax.fori_loop(0, _SC_ITERS, body, (z, z, z, z))
        codes_v[j] = accs[0] + accs[1] + accs[2] + accs[3]
    pltpu.sync_copy(codes_v, out_hbm.at[pl.ds(wid * np.int32(_SC_PER_TILE), _SC_PER_TILE)])


def _sc_seed_codes():
    mesh = plsc.VectorSubcoreMesh(core_axis_name="c", subcore_axis_name="s")
    fn = functools.partial(
        pl.kernel,
        mesh=mesh,
        out_type=jax.ShapeDtypeStruct((_SC_IMGS, 16), jnp.int32),
        scratch_types=[pltpu.VMEM((_SC_PER_TILE, 16), jnp.int32)],
    )(_sc_seed_kernel)
    return fn()


def _sc_combine_kernel(cvec_ref, code_ref, tot_ref):
    c = cvec_ref[...]
    code_ref[...] = jnp.sum(c, axis=1, keepdims=True)
    tot_ref[0, 0] = jnp.sum(c & np.int32(63))


def _apply_kernel(x_ref, code_ref, tot_a_ref, tot_b_ref, out_ref):
    tot = tot_a_ref[0, 0] + tot_b_ref[0, 0]
    dropped = (np.int32(_BS * _BS) * tot).astype(jnp.float32)
    scale = np.float32(_COUNTM) / (np.float32(_COUNTM) - dropped)
    codes = [code_ref[0, 0, i] for i in range(_APPLY_ROWS)]
    tilecnt = codes[0] & 63
    for s in codes[1:]:
        tilecnt += s & np.int32(63)

    @pl.when(tilecnt == 0)
    def _fast():
        out_ref[...] = x_ref[...] * scale

    @pl.when(tilecnt > 0)
    def _slow():
        oh = lax.broadcasted_iota(jnp.int32, (_H, _W), 0)
        ow = lax.broadcasted_iota(jnp.int32, (_H, _W), 1)
        for i in range(_APPLY_ROWS):
            s = codes[i]
            cnt = s & np.int32(63)
            q = lax.shift_right_logical(s, np.int32(6))
            hq = ((q.astype(jnp.float32) + np.float32(0.5)) * np.float32(1.0 / _WM)).astype(jnp.int32)
            h0 = jnp.where(cnt > 0, hq, np.int32(300))
            w0 = q - np.int32(_WM) * hq
            drop = (oh >= h0) & (oh < h0 + np.int32(_BS)) & (ow >= w0) & (ow < w0 + np.int32(_BS))
            out_ref[i] = x_ref[i] * jnp.where(drop, np.float32(0.0), scale)


def _dropblock_impl(x):
    xr = x.reshape(_NIMG, _H, _W)
    code_tc, tot_tc = pl.pallas_call(
        _seed_kernel,
        grid=(_SC_IMG0 // _SEED_IMGS,),
        out_specs=[
            pl.BlockSpec((1, 1, _SEED_IMGS), lambda i: (i, 0, 0), memory_space=pltpu.SMEM),
            pl.BlockSpec(memory_space=pltpu.SMEM),
        ],
        out_shape=[
            jax.ShapeDtypeStruct((_SC_IMG0 // _SEED_IMGS, 1, _SEED_IMGS), jnp.int32),
            jax.ShapeDtypeStruct((1, 1), jnp.int32),
        ],
    )()
    code_sc_vec = _sc_seed_codes()
    code_sc, tot_sc = pl.pallas_call(
        _sc_combine_kernel,
        grid=(1,),
        in_specs=[pl.BlockSpec((_SC_IMGS, 16), lambda i: (0, 0))],
        out_specs=[
            pl.BlockSpec((_SC_IMGS, 1), lambda i: (0, 0)),
            pl.BlockSpec(memory_space=pltpu.SMEM),
        ],
        out_shape=[
            jax.ShapeDtypeStruct((_SC_IMGS, 1), jnp.int32),
            jax.ShapeDtypeStruct((1, 1), jnp.int32),
        ],
    )(code_sc_vec)
    codes_all = jnp.concatenate([code_tc.reshape(_SC_IMG0), code_sc.reshape(_SC_IMGS)])
    code = codes_all.reshape(_NIMG // _APPLY_ROWS, 1, _APPLY_ROWS)
    out = pl.pallas_call(
        _apply_kernel,
        grid=(_NIMG // _APPLY_ROWS,),
        in_specs=[
            pl.BlockSpec((_APPLY_ROWS, _H, _W), lambda i: (i, 0, 0)),
            pl.BlockSpec((1, 1, _APPLY_ROWS), lambda i: (i, 0, 0), memory_space=pltpu.SMEM),
            pl.BlockSpec(memory_space=pltpu.SMEM),
            pl.BlockSpec(memory_space=pltpu.SMEM),
        ],
        out_specs=pl.BlockSpec((_APPLY_ROWS, _H, _W), lambda i: (i, 0, 0)),
        out_shape=jax.ShapeDtypeStruct((_NIMG, _H, _W), jnp.float32),
    )(xr, code, tot_tc, tot_sc)
    return out.reshape(_B, _C, _H, _W)


def kernel(x):
    return _dropblock_impl(x)


# speedup vs baseline: 2.4599x; 1.1587x over previous
import functools

import numpy as np
import jax
import jax.numpy as jnp
from jax import lax
from jax.experimental import pallas as pl
from jax.experimental.pallas import tpu as pltpu
from jax.experimental.pallas import tpu_sc as plsc

_B, _C, _H, _W = 8, 96, 224, 224
_BS = 7
_HM, _WM = _H - (_BS - 1), _W - (_BS - 1)
_NIMG = _B * _C
_LAT = _HM * _WM
_COUNTM = _B * _C * _H * _W

_KEEP_RATE = max(1.0 - 0.5 / 20000.0 * 1, 1.0 - 0.5)
_GAMMA = np.float32((1.0 - _KEEP_RATE) / _BS**2 * _W**2 / (_W - _BS + 1) ** 2)
_MTHRESH = int(np.ceil(np.float64(_GAMMA) * 2.0**23))


def _np_threefry2x32(ks, x0, x1):
    ks0, ks1 = np.uint32(ks[0]), np.uint32(ks[1])
    ks2 = ks0 ^ ks1 ^ np.uint32(0x1BD11BDA)
    x0 = (x0 + ks0).astype(np.uint32)
    x1 = (x1 + ks1).astype(np.uint32)
    rots = [(13, 15, 26, 6), (17, 29, 16, 24)]
    ksched = [(ks1, ks2), (ks2, ks0), (ks0, ks1), (ks1, ks2), (ks2, ks0)]
    for i in range(5):
        for r in rots[i % 2]:
            x0 = (x0 + x1).astype(np.uint32)
            x1 = ((x1 << np.uint32(r)) | (x1 >> np.uint32(32 - r))).astype(np.uint32)
            x1 = (x1 ^ x0).astype(np.uint32)
        a, b = ksched[i]
        x0 = (x0 + a).astype(np.uint32)
        x1 = (x1 + b + np.uint32(i + 1)).astype(np.uint32)
    return x0, x1


_FK0, _FK1 = _np_threefry2x32(
    (np.uint32(0), np.uint32(0)), np.array([0], np.uint32), np.array([1], np.uint32)
)
_KS0 = int(np.int32(np.uint32(_FK0[0])))
_KS1 = int(np.int32(np.uint32(_FK1[0])))
_KS2 = int(np.int32(np.uint32(_FK0[0]) ^ np.uint32(_FK1[0]) ^ np.uint32(0x1BD11BDA)))

_ROTS = ((13, 15, 26, 6), (17, 29, 16, 24))
_KSCHED = ((_KS1, _KS2), (_KS2, _KS0), (_KS0, _KS1), (_KS1, _KS2), (_KS2, _KS0))


def _rotl(x, r):
    return lax.shift_left(x, np.int32(r)) | lax.shift_right_logical(x, np.int32(32 - r))


def _threefry_bits(x1):
    x0 = jnp.full(x1.shape, _KS0, jnp.int32)
    x1 = x1 + np.int32(_KS1)
    for i in range(5):
        for r in _ROTS[i % 2]:
            x0 = x0 + x1
            x1 = _rotl(x1, r)
            x1 = x1 ^ x0
        a, b = _KSCHED[i]
        x0 = x0 + np.int32(a)
        x1 = x1 + np.int32(np.int32(b) + np.int32(i + 1))
    return x0 ^ x1


_APPLY_ROWS = 4
_SEED_IMGS = 4
_CROWS = 96
_CSIZE = _CROWS * 128
_NCHUNK = -(-_LAT // _CSIZE)


def _seed_kernel(code_ref, tot_ref):
    step = pl.program_id(0)

    @pl.when(step == 0)
    def _init():
        tot_ref[0, 0] = 0

    rr = lax.broadcasted_iota(jnp.int32, (_CROWS, 128), 0)
    l = lax.broadcasted_iota(jnp.int32, (_CROWS, 128), 1)
    qbase = rr * np.int32(128) + l
    wordbase = lax.shift_left(qbase, np.int32(6)) + 1

    for a in range(_SEED_IMGS):
        img = step * np.int32(_SEED_IMGS) + np.int32(a)
        base = img * np.int32(_LAT)

        def body(r, acc):
            off = r * np.int32(_CSIZE)
            bits = _threefry_bits(base + off + qbase)
            m = lax.shift_right_logical(bits, np.int32(9))
            isseed = (m < _MTHRESH) & (qbase < _LAT - off)
            word = wordbase + lax.shift_left(off, np.int32(6))
            return acc + jnp.where(isseed, word, np.int32(0))

        acc = lax.fori_loop(0, _NCHUNK, body, jnp.zeros((_CROWS, 128), jnp.int32))
        sa = jnp.sum(acc)
        code_ref[0, 0, a] = sa
        tot_ref[0, 0] += sa & np.int32(63)


_SC_IMG0 = 544
_SC_IMGS = _NIMG - _SC_IMG0
_SC_TILES = 32
_SC_PER_TILE = _SC_IMGS // _SC_TILES
_SC_UNROLL = 8
_SC_ITERS = -(-_LAT // (16 * _SC_UNROLL))


def _sc_seed_kernel(out_hbm, codes_v):
    wid = lax.axis_index("s") * np.int32(2) + lax.axis_index("c")
    li = lax.iota(jnp.int32, 16)
    for j in range(_SC_PER_TILE):
        img = np.int32(_SC_IMG0) + wid * np.int32(_SC_PER_TILE) + np.int32(j)
        base = img * np.int32(_LAT)

        def body(t, accs):
            qb = t * np.int32(16 * _SC_UNROLL)
            new = []
            for u in range(_SC_UNROLL):
                q = qb + np.int32(u * 16) + li
                bits = _threefry_bits(base + q)
                m = lax.shift_right_logical(bits, np.int32(9))
                isseed = (m < _MTHRESH) & (q < np.int32(_LAT))
                word = lax.shift_left(q, np.int32(6)) + 1
                new.append(accs[u] + jnp.where(isseed, word, np.int32(0)))
            return tuple(new)

        z = jnp.zeros((16,), jnp.int32)
        accs = lax.fori_loop(0, _SC_ITERS, body, (z,) * _SC_UNROLL)
        acc = accs[0]
        for t in accs[1:]:
            acc = acc + t
        codes_v[j] = acc
    for j in range(_SC_PER_TILE, 8):
        codes_v[j] = jnp.zeros((16,), jnp.int32)
    pltpu.sync_copy(codes_v, out_hbm.at[pl.ds(wid * np.int32(8), 8)])


def _sc_seed_codes():
    mesh = plsc.VectorSubcoreMesh(core_axis_name="c", subcore_axis_name="s")
    fn = functools.partial(
        pl.kernel,
        mesh=mesh,
        out_type=jax.ShapeDtypeStruct((_SC_TILES * 8, 16), jnp.int32),
        scratch_types=[pltpu.VMEM((8, 16), jnp.int32)],
    )(_sc_seed_kernel)
    raw = fn()
    return raw.reshape(_SC_TILES, 8, 16)[:, :_SC_PER_TILE].reshape(_SC_IMGS, 16)


def _sc_combine_kernel(cvec_ref, code_ref, tot_ref):
    c = cvec_ref[...]
    code_ref[...] = jnp.sum(c, axis=1, keepdims=True)
    tot_ref[0, 0] = jnp.sum(c & np.int32(63))


def _apply_kernel(x_ref, code_ref, tot_a_ref, tot_b_ref, out_ref):
    tot = tot_a_ref[0, 0] + tot_b_ref[0, 0]
    dropped = (np.int32(_BS * _BS) * tot).astype(jnp.float32)
    scale = np.float32(_COUNTM) / (np.float32(_COUNTM) - dropped)
    codes = [code_ref[0, 0, i] for i in range(_APPLY_ROWS)]
    tilecnt = codes[0] & 63
    for s in codes[1:]:
        tilecnt += s & np.int32(63)

    @pl.when(tilecnt == 0)
    def _fast():
        out_ref[...] = x_ref[...] * scale

    @pl.when(tilecnt > 0)
    def _slow():
        oh = lax.broadcasted_iota(jnp.int32, (_H, _W), 0)
        ow = lax.broadcasted_iota(jnp.int32, (_H, _W), 1)
        for i in range(_APPLY_ROWS):
            s = codes[i]
            cnt = s & np.int32(63)
            q = lax.shift_right_logical(s, np.int32(6))
            hq = ((q.astype(jnp.float32) + np.float32(0.5)) * np.float32(1.0 / _WM)).astype(jnp.int32)
            h0 = jnp.where(cnt > 0, hq, np.int32(300))
            w0 = q - np.int32(_WM) * hq
            drop = (oh >= h0) & (oh < h0 + np.int32(_BS)) & (ow >= w0) & (ow < w0 + np.int32(_BS))
            out_ref[i] = x_ref[i] * jnp.where(drop, np.float32(0.0), scale)


def _dropblock_impl(x):
    xr = x.reshape(_NIMG, _H, _W)
    code_tc, tot_tc = pl.pallas_call(
        _seed_kernel,
        grid=(_SC_IMG0 // _SEED_IMGS,),
        out_specs=[
            pl.BlockSpec((1, 1, _SEED_IMGS), lambda i: (i, 0, 0), memory_space=pltpu.SMEM),
            pl.BlockSpec(memory_space=pltpu.SMEM),
        ],
        out_shape=[
            jax.ShapeDtypeStruct((_SC_IMG0 // _SEED_IMGS, 1, _SEED_IMGS), jnp.int32),
            jax.ShapeDtypeStruct((1, 1), jnp.int32),
        ],
    )()
    code_sc_vec = _sc_seed_codes()
    code_sc, tot_sc = pl.pallas_call(
        _sc_combine_kernel,
        grid=(1,),
        in_specs=[pl.BlockSpec((_SC_IMGS, 16), lambda i: (0, 0))],
        out_specs=[
            pl.BlockSpec((_SC_IMGS, 1), lambda i: (0, 0)),
            pl.BlockSpec(memory_space=pltpu.SMEM),
        ],
        out_shape=[
            jax.ShapeDtypeStruct((_SC_IMGS, 1), jnp.int32),
            jax.ShapeDtypeStruct((1, 1), jnp.int32),
        ],
    )(code_sc_vec)
    codes_all = jnp.concatenate([code_tc.reshape(_SC_IMG0), code_sc.reshape(_SC_IMGS)])
    code = codes_all.reshape(_NIMG // _APPLY_ROWS, 1, _APPLY_ROWS)
    out = pl.pallas_call(
        _apply_kernel,
        grid=(_NIMG // _APPLY_ROWS,),
        in_specs=[
            pl.BlockSpec((_APPLY_ROWS, _H, _W), lambda i: (i, 0, 0)),
            pl.BlockSpec((1, 1, _APPLY_ROWS), lambda i: (i, 0, 0), memory_space=pltpu.SMEM),
            pl.BlockSpec(memory_space=pltpu.SMEM),
            pl.BlockSpec(memory_space=pltpu.SMEM),
        ],
        out_specs=pl.BlockSpec((_APPLY_ROWS, _H, _W), lambda i: (i, 0, 0)),
        out_shape=jax.ShapeDtypeStruct((_NIMG, _H, _W), jnp.float32),
    )(xr, code, tot_tc, tot_sc)
    return out.reshape(_B, _C, _H, _W)


def kernel(x):
    return _dropblock_impl(x)
